# speedup vs baseline: 1.0089x; 1.0089x over previous
import functools
import jax
import jax.numpy as jnp
import numpy as np
from jax.experimental import pallas as pl
from jax.experimental.pallas import tpu as pltpu

N = 16384
K = 20
NUM_GRAPHS = 16
CHUNK = 2048


def _knn_idx(x, batch, k):
    n = x.shape[0]
    sq = jnp.sum(x * x, axis=1)
    outs = []
    for s in range(0, n, CHUNK):
        rows = jnp.arange(s, s + CHUNK)
        d = sq[rows][:, None] + sq[None, :] - 2.0 * (x[rows] @ x.T)
        d = jnp.where(batch[rows][:, None] != batch[None, :], jnp.inf, d)
        d = d.at[jnp.arange(CHUNK), rows].set(jnp.inf)
        _, idx = jax.lax.top_k(-d, k)
        outs.append(idx)
    return jnp.concatenate(outs, axis=0)


def _ec_body(c_ref, g_ref, w2_ref, b2_ref, w3_ref, b3_ref, o_ref):
    c = c_ref[...]
    acc = None
    for k in range(K):
        h = jnp.maximum(c + g_ref[k], 0.0)
        h = jnp.maximum(
            jnp.dot(h, w2_ref[...], preferred_element_type=jnp.float32)
            + b2_ref[...], 0.0)
        h = jnp.maximum(
            jnp.dot(h, w3_ref[...], preferred_element_type=jnp.float32)
            + b3_ref[...], 0.0)
        acc = h if acc is None else jnp.maximum(acc, h)
    o_ref[...] = acc


def _edge_conv(C, G, W2, b2, W3, b3, R=512):
    Din = C.shape[1]
    Dmid = W2.shape[1]
    Dout = W3.shape[1]
    b2 = b2.reshape(1, Dmid)
    b3 = b3.reshape(1, Dout)
    return pl.pallas_call(
        _ec_body,
        grid=(N // R,),
        in_specs=[
            pl.BlockSpec((R, Din), lambda r: (r, 0)),
            pl.BlockSpec((K, R, Din), lambda r: (0, r, 0)),
            pl.BlockSpec((Din, Dmid), lambda r: (0, 0)),
            pl.BlockSpec((1, Dmid), lambda r: (0, 0)),
            pl.BlockSpec((Dmid, Dout), lambda r: (0, 0)),
            pl.BlockSpec((1, Dout), lambda r: (0, 0)),
        ],
        out_specs=pl.BlockSpec((R, Dout), lambda r: (r, 0)),
        out_shape=jax.ShapeDtypeStruct((N, Dout), jnp.float32),
    )(C, G, W2, b2, W3, b3)


def kernel(pos, batch, W1a, b1a, W1b, b1b, W1c, b1c, W2a, b2a, W2b, b2b,
           W2c, b2c, W0, b0, Wl1, bl1, Wl2, bl2, Wl3, bl3):
    nbr1 = _knn_idx(pos, batch, K)
    d_in = pos.shape[1]
    P1 = pos @ W1a[:d_in] + b1a
    Q1 = pos @ W1a[d_in:]
    C1 = P1 - Q1
    G1 = Q1[nbr1.T]
    x1 = _edge_conv(C1, G1, W1b, b1b, W1c, b1c)

    nbr2 = _knn_idx(x1, batch, K)
    P2 = x1 @ W2a[:64] + b2a
    Q2 = x1 @ W2a[64:]
    C2 = P2 - Q2
    G2 = Q2[nbr2.T]
    x2 = _edge_conv(C2, G2, W2b, b2b, W2c, b2c)

    y = jax.nn.relu(x2 @ W0 + b0)
    y = jax.ops.segment_max(y, batch, num_segments=NUM_GRAPHS)
    y = jax.nn.relu(y @ Wl1 + bl1)
    y = jax.nn.relu(y @ Wl2 + bl2)
    y = y @ Wl3 + bl3
    return jax.nn.log_softmax(y, axis=-1)


# speedup vs baseline: 10.9509x; 10.8548x over previous
import functools
import jax
import jax.numpy as jnp
import numpy as np
from jax.experimental import pallas as pl
from jax.experimental.pallas import tpu as pltpu

N = 16384
K = 20
NUM_GRAPHS = 16
CHUNK = 2048


_KNN_R = 256
_KNN_CT = 1024


def _knn_body(lo_ref, hi_ref, x_ref, xT_ref, bT_ref, bC_ref, out_ref, ds_ref):
    R, CT = _KNN_R, _KNN_CT
    r = pl.program_id(0)
    lo_t = lo_ref[r]
    hi_t = hi_ref[r]

    xr = x_ref[pl.ds(r * R, R), :]
    sq_r = jnp.sum(xr * xr, axis=1, keepdims=True)
    b_r = bT_ref[...]
    row_gid = r * R + jax.lax.broadcasted_iota(jnp.int32, (R, 1), 0)

    inf = jnp.float32(jnp.inf)
    big_i = jnp.int32(2**30)

    def fill_tile(t, _):
        c0 = t * CT
        xc = xT_ref[:, pl.ds(c0, CT)]
        dot = jnp.dot(xr, xc, preferred_element_type=jnp.float32)
        sq_c = jnp.sum(xc * xc, axis=0, keepdims=True)
        d = sq_r + sq_c - 2.0 * dot
        b_c = bC_ref[:, pl.ds(c0, CT)]
        col = c0 + jax.lax.broadcasted_iota(jnp.int32, (R, CT), 1)
        d = jnp.where((b_r != b_c) | (col == row_gid), inf, d)
        ds_ref[:, pl.ds(c0, CT)] = d
        return 0

    jax.lax.fori_loop(lo_t, hi_t, fill_tile, 0)

    def extract(kk, _):
        def tile_min(t, m):
            dt = ds_ref[:, pl.ds(t * CT, CT)]
            return jnp.minimum(m, jnp.min(dt, axis=1, keepdims=True))

        m = jax.lax.fori_loop(lo_t, hi_t, tile_min,
                              jnp.full((R, 1), inf, jnp.float32))

        def tile_arg(t, ix):
            c0 = t * CT
            dt = ds_ref[:, pl.ds(c0, CT)]
            col = c0 + jax.lax.broadcasted_iota(jnp.int32, (R, CT), 1)
            cand = jnp.where(dt == m, col, big_i)
            return jnp.minimum(ix, jnp.min(cand, axis=1, keepdims=True))

        idx = jax.lax.fori_loop(lo_t, hi_t, tile_arg,
                                jnp.full((R, 1), big_i, jnp.int32))

        def tile_inval(t, _):
            c0 = t * CT
            dt = ds_ref[:, pl.ds(c0, CT)]
            col = c0 + jax.lax.broadcasted_iota(jnp.int32, (R, CT), 1)
            ds_ref[:, pl.ds(c0, CT)] = jnp.where(col == idx, inf, dt)
            return 0

        jax.lax.fori_loop(lo_t, hi_t, tile_inval, 0)
        out_ref[:, kk : kk + 1] = jnp.minimum(idx, jnp.int32(N - 1))

    for kk in range(K):
        extract(kk, 0)


def _knn_idx(x, batch):
    n, d = x.shape
    dp = 8 if d < 8 else d
    if d != dp:
        x = jnp.pad(x, ((0, 0), (0, dp - d)))
    xT = x.T
    bT = batch.reshape(n, 1)
    bC = batch.reshape(1, n)

    nb = n // _KNN_R
    seg_start = jnp.searchsorted(batch, jnp.arange(NUM_GRAPHS), side="left")
    seg_end = jnp.searchsorted(batch, jnp.arange(NUM_GRAPHS), side="right")
    b_lo = batch[:: _KNN_R]
    b_hi = batch[_KNN_R - 1 :: _KNN_R]
    lo_t = (seg_start[b_lo] // _KNN_CT).astype(jnp.int32)
    hi_t = ((seg_end[b_hi] + _KNN_CT - 1) // _KNN_CT).astype(jnp.int32)

    out = pl.pallas_call(
        _knn_body,
        grid_spec=pltpu.PrefetchScalarGridSpec(
            num_scalar_prefetch=2,
            grid=(nb,),
            in_specs=[
                pl.BlockSpec((n, dp), lambda r, lo, hi: (0, 0)),
                pl.BlockSpec((dp, n), lambda r, lo, hi: (0, 0)),
                pl.BlockSpec((_KNN_R, 1), lambda r, lo, hi: (r, 0)),
                pl.BlockSpec((1, n), lambda r, lo, hi: (0, 0)),
            ],
            out_specs=pl.BlockSpec((_KNN_R, 32), lambda r, lo, hi: (r, 0)),
            scratch_shapes=[pltpu.VMEM((_KNN_R, n), jnp.float32)],
        ),
        out_shape=jax.ShapeDtypeStruct((n, 32), jnp.int32),
    )(lo_t, hi_t, x, xT, bT, bC)
    return out[:, :K].T


def _ec_body(c_ref, g_ref, w2_ref, b2_ref, w3_ref, b3_ref, o_ref):
    c = c_ref[...]
    acc = None
    for k in range(K):
        h = jnp.maximum(c + g_ref[k], 0.0)
        h = jnp.maximum(
            jnp.dot(h, w2_ref[...], preferred_element_type=jnp.float32)
            + b2_ref[...], 0.0)
        h = jnp.maximum(
            jnp.dot(h, w3_ref[...], preferred_element_type=jnp.float32)
            + b3_ref[...], 0.0)
        acc = h if acc is None else jnp.maximum(acc, h)
    o_ref[...] = acc


def _edge_conv(C, G, W2, b2, W3, b3, R=512):
    Din = C.shape[1]
    Dmid = W2.shape[1]
    Dout = W3.shape[1]
    b2 = b2.reshape(1, Dmid)
    b3 = b3.reshape(1, Dout)
    return pl.pallas_call(
        _ec_body,
        grid=(N // R,),
        in_specs=[
            pl.BlockSpec((R, Din), lambda r: (r, 0)),
            pl.BlockSpec((K, R, Din), lambda r: (0, r, 0)),
            pl.BlockSpec((Din, Dmid), lambda r: (0, 0)),
            pl.BlockSpec((1, Dmid), lambda r: (0, 0)),
            pl.BlockSpec((Dmid, Dout), lambda r: (0, 0)),
            pl.BlockSpec((1, Dout), lambda r: (0, 0)),
        ],
        out_specs=pl.BlockSpec((R, Dout), lambda r: (r, 0)),
        out_shape=jax.ShapeDtypeStruct((N, Dout), jnp.float32),
    )(C, G, W2, b2, W3, b3)


def kernel(pos, batch, W1a, b1a, W1b, b1b, W1c, b1c, W2a, b2a, W2b, b2b,
           W2c, b2c, W0, b0, Wl1, bl1, Wl2, bl2, Wl3, bl3):
    nbr1 = _knn_idx(pos, batch)
    d_in = pos.shape[1]
    P1 = pos @ W1a[:d_in] + b1a
    Q1 = pos @ W1a[d_in:]
    C1 = P1 - Q1
    G1 = Q1[nbr1]
    x1 = _edge_conv(C1, G1, W1b, b1b, W1c, b1c)

    nbr2 = _knn_idx(x1, batch)
    P2 = x1 @ W2a[:64] + b2a
    Q2 = x1 @ W2a[64:]
    C2 = P2 - Q2
    G2 = Q2[nbr2]
    x2 = _edge_conv(C2, G2, W2b, b2b, W2c, b2c)

    y = jax.nn.relu(x2 @ W0 + b0)
    y = jax.ops.segment_max(y, batch, num_segments=NUM_GRAPHS)
    y = jax.nn.relu(y @ Wl1 + bl1)
    y = jax.nn.relu(y @ Wl2 + bl2)
    y = y @ Wl3 + bl3
    return jax.nn.log_softmax(y, axis=-1)


# speedup vs baseline: 11.4081x; 1.0417x over previous
import functools
import jax
import jax.numpy as jnp
import numpy as np
from jax.experimental import pallas as pl
from jax.experimental.pallas import tpu as pltpu

N = 16384
K = 20
NUM_GRAPHS = 16
CHUNK = 2048


_KNN_R = 256
_KNN_CT = 512
_KNN_NT = N // _KNN_CT


def _knn_body(lo_ref, hi_ref, x_ref, xT_ref, bT_ref, bC_ref, wc_ref, wq_ref,
              bc_ref, out_ref, c_out_ref, q_out_ref, ds_ref):
    R, CT = _KNN_R, _KNN_CT
    r = pl.program_id(0)
    lo_t = lo_ref[r]
    hi_t = hi_ref[r]

    xr = x_ref[pl.ds(r * R, R), :]
    sq_r = jnp.sum(xr * xr, axis=1, keepdims=True)
    b_r = bT_ref[...]
    row_gid = r * R + jax.lax.broadcasted_iota(jnp.int32, (R, 1), 0)

    c_out_ref[...] = jnp.dot(xr, wc_ref[...],
                             preferred_element_type=jnp.float32) + bc_ref[...]
    q_out_ref[...] = jnp.dot(xr, wq_ref[...],
                             preferred_element_type=jnp.float32)

    inf = jnp.float32(jnp.inf)
    big_i = jnp.int32(2**30)
    lane_t = jax.lax.broadcasted_iota(jnp.int32, (R, _KNN_NT), 1)

    def fill_tile(t, M):
        c0 = t * CT
        xc = xT_ref[:, pl.ds(c0, CT)]
        dot = jnp.dot(xr, xc, preferred_element_type=jnp.float32)
        sq_c = jnp.sum(xc * xc, axis=0, keepdims=True)
        d = sq_r + sq_c - 2.0 * dot
        b_c = bC_ref[:, pl.ds(c0, CT)]
        col = c0 + jax.lax.broadcasted_iota(jnp.int32, (R, CT), 1)
        d = jnp.where((b_r != b_c) | (col == row_gid), inf, d)
        ds_ref[:, pl.ds(c0, CT)] = d
        mt = jnp.min(d, axis=1, keepdims=True)
        return jnp.where(lane_t == t, mt, M)

    M = jax.lax.fori_loop(lo_t, hi_t, fill_tile,
                          jnp.full((R, _KNN_NT), inf, jnp.float32))

    for kk in range(K):
        m = jnp.min(M, axis=1, keepdims=True)

        def tile_scan(t, carry):
            idx, Mc = carry
            c0 = t * CT
            dt = ds_ref[:, pl.ds(c0, CT)]
            col = c0 + jax.lax.broadcasted_iota(jnp.int32, (R, CT), 1)
            cand = jnp.where(dt == m, col, big_i)
            idx = jnp.minimum(idx, jnp.min(cand, axis=1, keepdims=True))
            mt = jnp.min(jnp.where(dt > m, dt, inf), axis=1, keepdims=True)
            Mc = jnp.where(lane_t == t, mt, Mc)
            return idx, Mc

        idx, M = jax.lax.fori_loop(
            lo_t, hi_t, tile_scan,
            (jnp.full((R, 1), big_i, jnp.int32), M))
        out_ref[:, kk : kk + 1] = jnp.minimum(idx, jnp.int32(N - 1))


def _knn_prep(x, batch, W, b):
    n, d = x.shape
    F = W.shape[1]
    Wc = W[:d] - W[d:]
    Wq = W[d:]
    dp = 8 if d < 8 else d
    if d != dp:
        x = jnp.pad(x, ((0, 0), (0, dp - d)))
        Wc = jnp.pad(Wc, ((0, dp - d), (0, 0)))
        Wq = jnp.pad(Wq, ((0, dp - d), (0, 0)))
    xT = x.T
    bT = batch.reshape(n, 1)
    bC = batch.reshape(1, n)

    nb = n // _KNN_R
    seg_start = jnp.searchsorted(batch, jnp.arange(NUM_GRAPHS), side="left")
    seg_end = jnp.searchsorted(batch, jnp.arange(NUM_GRAPHS), side="right")
    b_lo = batch[:: _KNN_R]
    b_hi = batch[_KNN_R - 1 :: _KNN_R]
    lo_t = (seg_start[b_lo] // _KNN_CT).astype(jnp.int32)
    hi_t = ((seg_end[b_hi] + _KNN_CT - 1) // _KNN_CT).astype(jnp.int32)

    nbr, C, Q = pl.pallas_call(
        _knn_body,
        grid_spec=pltpu.PrefetchScalarGridSpec(
            num_scalar_prefetch=2,
            grid=(nb,),
            in_specs=[
                pl.BlockSpec((n, dp), lambda r, lo, hi: (0, 0)),
                pl.BlockSpec((dp, n), lambda r, lo, hi: (0, 0)),
                pl.BlockSpec((_KNN_R, 1), lambda r, lo, hi: (r, 0)),
                pl.BlockSpec((1, n), lambda r, lo, hi: (0, 0)),
                pl.BlockSpec((dp, F), lambda r, lo, hi: (0, 0)),
                pl.BlockSpec((dp, F), lambda r, lo, hi: (0, 0)),
                pl.BlockSpec((1, F), lambda r, lo, hi: (0, 0)),
            ],
            out_specs=[
                pl.BlockSpec((_KNN_R, 32), lambda r, lo, hi: (r, 0)),
                pl.BlockSpec((_KNN_R, F), lambda r, lo, hi: (r, 0)),
                pl.BlockSpec((_KNN_R, F), lambda r, lo, hi: (r, 0)),
            ],
            scratch_shapes=[pltpu.VMEM((_KNN_R, n), jnp.float32)],
        ),
        out_shape=[
            jax.ShapeDtypeStruct((n, 32), jnp.int32),
            jax.ShapeDtypeStruct((n, F), jnp.float32),
            jax.ShapeDtypeStruct((n, F), jnp.float32),
        ],
    )(lo_t, hi_t, x, xT, bT, bC, Wc, Wq, b.reshape(1, F))
    return nbr[:, :K].T, C, Q


def _ec_body(c_ref, g_ref, w2_ref, b2_ref, w3_ref, b3_ref, o_ref):
    c = c_ref[...]
    acc = None
    for k in range(K):
        h = jnp.maximum(c + g_ref[k], 0.0)
        h = jnp.maximum(
            jnp.dot(h, w2_ref[...], preferred_element_type=jnp.float32)
            + b2_ref[...], 0.0)
        h = jnp.maximum(
            jnp.dot(h, w3_ref[...], preferred_element_type=jnp.float32)
            + b3_ref[...], 0.0)
        acc = h if acc is None else jnp.maximum(acc, h)
    o_ref[...] = acc


def _edge_conv(C, G, W2, b2, W3, b3, R=512):
    Din = C.shape[1]
    Dmid = W2.shape[1]
    Dout = W3.shape[1]
    b2 = b2.reshape(1, Dmid)
    b3 = b3.reshape(1, Dout)
    return pl.pallas_call(
        _ec_body,
        grid=(N // R,),
        in_specs=[
            pl.BlockSpec((R, Din), lambda r: (r, 0)),
            pl.BlockSpec((K, R, Din), lambda r: (0, r, 0)),
            pl.BlockSpec((Din, Dmid), lambda r: (0, 0)),
            pl.BlockSpec((1, Dmid), lambda r: (0, 0)),
            pl.BlockSpec((Dmid, Dout), lambda r: (0, 0)),
            pl.BlockSpec((1, Dout), lambda r: (0, 0)),
        ],
        out_specs=pl.BlockSpec((R, Dout), lambda r: (r, 0)),
        out_shape=jax.ShapeDtypeStruct((N, Dout), jnp.float32),
    )(C, G, W2, b2, W3, b3)


def kernel(pos, batch, W1a, b1a, W1b, b1b, W1c, b1c, W2a, b2a, W2b, b2b,
           W2c, b2c, W0, b0, Wl1, bl1, Wl2, bl2, Wl3, bl3):
    nbr1, C1, Q1 = _knn_prep(pos, batch, W1a, b1a)
    G1 = Q1[nbr1]
    x1 = _edge_conv(C1, G1, W1b, b1b, W1c, b1c)

    nbr2, C2, Q2 = _knn_prep(x1, batch, W2a, b2a)
    G2 = Q2[nbr2]
    x2 = _edge_conv(C2, G2, W2b, b2b, W2c, b2c)

    y = jax.nn.relu(x2 @ W0 + b0)
    y = jax.ops.segment_max(y, batch, num_segments=NUM_GRAPHS)
    y = jax.nn.relu(y @ Wl1 + bl1)
    y = jax.nn.relu(y @ Wl2 + bl2)
    y = y @ Wl3 + bl3
    return jax.nn.log_softmax(y, axis=-1)
